# baseline (speedup 1.0000x reference)
import functools

import jax
import jax.numpy as jnp
from jax import lax
from jax.experimental import pallas as pl
from jax.experimental.pallas import tpu as pltpu
from jax.experimental.pallas import tpu_sc as plsc

_NUM_SAMPLES = 1024
_NUM_SEG = 9
_FEAT = 3072
_NUM_CLASSES = 20
_NLANE = 16
_NWORK = 32
_PPW = _NUM_SAMPLES // _NWORK

_DN = (((1,), (1,)), ((), ()))


def _dot_t(a, w):
    return jax.lax.dot_general(a, w, _DN, preferred_element_type=jnp.float32)


def _sc_pool_kernel(x_hbm, out_hbm, xv, ob, isems, osems):
    F = _FEAT
    wid = lax.axis_index("s") * 2 + lax.axis_index("c")
    base = wid * _PPW

    def copy_in(i, slot):
        return pltpu.make_async_copy(
            x_hbm.at[pl.ds((base + i) * _NUM_SEG * F, _NUM_SEG * F)],
            xv.at[slot], isems.at[slot])

    def copy_out(i, slot):
        return pltpu.make_async_copy(
            ob.at[slot], out_hbm.at[pl.ds((base + i) * 3 * F, 3 * F)],
            osems.at[slot])

    copy_in(0, 0).start()
    copy_in(1, 1).start()

    def pool_one(i, slot):
        copy_in(i, slot).wait()

        @pl.when(i >= 2)
        def _():
            copy_out(i - 2, slot).wait()

        def feat(j, _):
            o = j * _NLANE
            sv = [xv[slot, pl.ds(s * F + o, _NLANE)] for s in range(_NUM_SEG)]
            s0 = (sv[0] + sv[1]) * 0.5
            cs = (sv[2] + sv[3] + sv[4] + sv[5] + sv[6]) * 0.2
            en = (sv[7] + sv[8]) * 0.5
            ob[slot, pl.ds(o, _NLANE)] = s0
            ob[slot, pl.ds(F + o, _NLANE)] = cs
            ob[slot, pl.ds(2 * F + o, _NLANE)] = en
            return ()

        lax.fori_loop(0, F // _NLANE, feat, (), unroll=4)
        copy_out(i, slot).start()

        @pl.when(i + 2 < _PPW)
        def _():
            copy_in(i + 2, slot).start()

    def body(k, _):
        pool_one(2 * k, 0)
        pool_one(2 * k + 1, 1)
        return ()

    lax.fori_loop(0, _PPW // 2, body, ())
    copy_out(_PPW - 2, 0).wait()
    copy_out(_PPW - 1, 1).wait()


@jax.jit
def _sc_pool(x):
    mesh = plsc.VectorSubcoreMesh(core_axis_name="c", subcore_axis_name="s")
    kfn = functools.partial(
        pl.kernel, mesh=mesh,
        out_type=jax.ShapeDtypeStruct((_NUM_SAMPLES * 3 * _FEAT,), jnp.float32),
        scratch_types=[
            pltpu.VMEM((2, _NUM_SEG * _FEAT), jnp.float32),
            pltpu.VMEM((2, 3 * _FEAT), jnp.float32),
            pltpu.SemaphoreType.DMA((2,)),
            pltpu.SemaphoreType.DMA((2,)),
        ],
    )(_sc_pool_kernel)
    return kfn(x.reshape(-1)).reshape(_NUM_SAMPLES, 3 * _FEAT)


def _tc_dots_kernel(p_ref, sf_ref, wa_ref, ba_ref, wc_ref, bc_ref,
                    wr_ref, br_ref, act_ref, comp_ref, reg_ref):
    F = _FEAT
    sf = sf_ref[...]
    start = p_ref[:, 0:F] * sf[:, 0:1]
    course = p_ref[:, F:2 * F]
    end = p_ref[:, 2 * F:3 * F] * sf[:, 1:2]
    act_ref[...] = _dot_t(course, wa_ref[...]) + ba_ref[...]
    comp_ref[...] = (_dot_t(start, wc_ref[:, 0:F])
                     + _dot_t(course, wc_ref[:, F:2 * F])
                     + _dot_t(end, wc_ref[:, 2 * F:3 * F]) + bc_ref[...])
    reg_ref[...] = (_dot_t(start, wr_ref[:, 0:F])
                    + _dot_t(course, wr_ref[:, F:2 * F])
                    + _dot_t(end, wr_ref[:, 2 * F:3 * F]) + br_ref[...])


@functools.partial(jax.jit, static_argnames=("block",))
def _tc_dots(pooled, sf, W_act, b_act, W_comp, b_comp, W_reg, b_reg, block=128):
    grid = _NUM_SAMPLES // block
    nw = lambda i: (0, 0)
    outs = pl.pallas_call(
        _tc_dots_kernel,
        grid=(grid,),
        in_specs=[
            pl.BlockSpec((block, 3 * _FEAT), lambda i: (i, 0)),
            pl.BlockSpec((block, 2), lambda i: (i, 0)),
            pl.BlockSpec(W_act.shape, nw),
            pl.BlockSpec(b_act.shape, nw),
            pl.BlockSpec(W_comp.shape, nw),
            pl.BlockSpec(b_comp.shape, nw),
            pl.BlockSpec(W_reg.shape, nw),
            pl.BlockSpec(b_reg.shape, nw),
        ],
        out_specs=[
            pl.BlockSpec((block, _NUM_CLASSES + 1), lambda i: (i, 0)),
            pl.BlockSpec((block, _NUM_CLASSES), lambda i: (i, 0)),
            pl.BlockSpec((block, _NUM_CLASSES * 2), lambda i: (i, 0)),
        ],
        out_shape=[
            jax.ShapeDtypeStruct((_NUM_SAMPLES, _NUM_CLASSES + 1), jnp.float32),
            jax.ShapeDtypeStruct((_NUM_SAMPLES, _NUM_CLASSES), jnp.float32),
            jax.ShapeDtypeStruct((_NUM_SAMPLES, _NUM_CLASSES * 2), jnp.float32),
        ],
    )(pooled, sf, W_act, b_act, W_comp, b_comp, W_reg, b_reg)
    return outs


def kernel(x, scale_factors, W_act, b_act, W_comp, b_comp, W_reg, b_reg):
    pooled = _sc_pool(x)
    act, comp, reg = _tc_dots(pooled, scale_factors,
                              W_act, b_act.reshape(1, -1),
                              W_comp, b_comp.reshape(1, -1),
                              W_reg, b_reg.reshape(1, -1))
    return (act, comp, reg.reshape(-1, _NUM_CLASSES, 2))


# speedup vs baseline: 5.1799x; 5.1799x over previous
import functools

import jax
import jax.numpy as jnp
from jax.experimental import pallas as pl
from jax.experimental.pallas import tpu as pltpu

_NUM_SAMPLES = 1024
_NUM_SEG = 9
_FEAT = 3072
_NUM_CLASSES = 20

_DN = (((1,), (1,)), ((), ()))


def _dot_t(a, w):
    return jax.lax.dot_general(a, w, _DN, preferred_element_type=jnp.float32)


def _fused_kernel(x_ref, sf_ref, wa_ref, ba_ref, wc_ref, bc_ref, wr_ref, br_ref,
                  act_ref, comp_ref, reg_ref):
    F = _FEAT
    xb = x_ref[...]
    sf = sf_ref[...]
    xr = xb.reshape(-1, _NUM_SEG, F)
    start = (xr[:, 0, :] + xr[:, 1, :]) * (sf[:, 0:1] * 0.5)
    course = (xr[:, 2, :] + xr[:, 3, :] + xr[:, 4, :]
              + xr[:, 5, :] + xr[:, 6, :]) * 0.2
    end = (xr[:, 7, :] + xr[:, 8, :]) * (sf[:, 1:2] * 0.5)
    act_ref[...] = _dot_t(course, wa_ref[...]) + ba_ref[...]
    comp_ref[...] = (_dot_t(start, wc_ref[:, 0:F])
                     + _dot_t(course, wc_ref[:, F:2 * F])
                     + _dot_t(end, wc_ref[:, 2 * F:3 * F]) + bc_ref[...])
    reg_ref[...] = (_dot_t(start, wr_ref[:, 0:F])
                    + _dot_t(course, wr_ref[:, F:2 * F])
                    + _dot_t(end, wr_ref[:, 2 * F:3 * F]) + br_ref[...])


@functools.partial(jax.jit, static_argnames=("block",))
def _run(x, sf, W_act, b_act, W_comp, b_comp, W_reg, b_reg, block=128):
    grid = _NUM_SAMPLES // block
    nw = lambda i: (0, 0)
    outs = pl.pallas_call(
        _fused_kernel,
        grid=(grid,),
        in_specs=[
            pl.BlockSpec((block * _NUM_SEG, _FEAT), lambda i: (i, 0)),
            pl.BlockSpec((block, 2), lambda i: (i, 0)),
            pl.BlockSpec(W_act.shape, nw),
            pl.BlockSpec(b_act.shape, nw),
            pl.BlockSpec(W_comp.shape, nw),
            pl.BlockSpec(b_comp.shape, nw),
            pl.BlockSpec(W_reg.shape, nw),
            pl.BlockSpec(b_reg.shape, nw),
        ],
        out_specs=[
            pl.BlockSpec((block, _NUM_CLASSES + 1), lambda i: (i, 0)),
            pl.BlockSpec((block, _NUM_CLASSES), lambda i: (i, 0)),
            pl.BlockSpec((block, _NUM_CLASSES * 2), lambda i: (i, 0)),
        ],
        out_shape=[
            jax.ShapeDtypeStruct((_NUM_SAMPLES, _NUM_CLASSES + 1), jnp.float32),
            jax.ShapeDtypeStruct((_NUM_SAMPLES, _NUM_CLASSES), jnp.float32),
            jax.ShapeDtypeStruct((_NUM_SAMPLES, _NUM_CLASSES * 2), jnp.float32),
        ],
        compiler_params=pltpu.CompilerParams(
            dimension_semantics=("arbitrary",)),
    )(x, sf, W_act, b_act, W_comp, b_comp, W_reg, b_reg)
    return outs


def kernel(x, scale_factors, W_act, b_act, W_comp, b_comp, W_reg, b_reg):
    act, comp, reg = _run(x, scale_factors,
                          W_act, b_act.reshape(1, -1),
                          W_comp, b_comp.reshape(1, -1),
                          W_reg, b_reg.reshape(1, -1))
    return (act, comp, reg.reshape(-1, _NUM_CLASSES, 2))
